# baseline (speedup 1.0000x reference)
import functools

import jax
import jax.numpy as jnp
from jax import lax
from jax.experimental import pallas as pl
from jax.experimental.pallas import tpu as pltpu
from jax.experimental.pallas import tpu_sc as plsc

_B = 16384
_K = 3
_H = 2048
_N = _B * _K

_info = plsc.get_sparse_core_info()
_NC, _NS, _L = _info.num_cores, _info.num_subcores, _info.num_lanes
_NW = _NC * _NS
_BPW = _N // _NW
_CH = 16
_NCHUNK = _BPW // _CH

_mesh = plsc.VectorSubcoreMesh(core_axis_name="c", subcore_axis_name="s")


@functools.partial(
    pl.kernel,
    mesh=_mesh,
    out_type=jax.ShapeDtypeStruct((_N, _H), jnp.float32),
    scratch_types=[
        pltpu.VMEM((_BPW,), jnp.int32),
        pltpu.VMEM((_CH, _H), jnp.float32),
        pltpu.SemaphoreType.DMA,
    ],
)
def _lookup(idx_hbm, table_hbm, out_hbm, idx_v, buf, sem):
    wid = lax.axis_index("s") * _NC + lax.axis_index("c")
    base = wid * _BPW
    pltpu.sync_copy(idx_hbm.at[pl.ds(base, _BPW)], idx_v)

    def _remap(i, carry):
        v = idx_v[pl.ds(i * _L, _L)]
        idx_v[pl.ds(i * _L, _L)] = jnp.where(v < 0, 2, v)
        return carry

    lax.fori_loop(0, _BPW // _L, _remap, 0)

    def _chunk(c, carry):
        pltpu.async_copy(
            table_hbm.at[idx_v.at[pl.ds(c * _CH, _CH)]], buf, sem
        ).wait()
        pltpu.sync_copy(buf, out_hbm.at[pl.ds(base + c * _CH, _CH)])
        return carry

    lax.fori_loop(0, _NCHUNK, _chunk, 0)


def kernel(arm_labels, embedding_weight):
    idx = arm_labels.reshape(_N).astype(jnp.int32)
    out = _lookup(idx, embedding_weight)
    return out.reshape(_B, _K, _H)


# speedup vs baseline: 2.3241x; 2.3241x over previous
import functools

import jax
import jax.numpy as jnp
from jax import lax
from jax.experimental import pallas as pl
from jax.experimental.pallas import tpu as pltpu
from jax.experimental.pallas import tpu_sc as plsc

_B = 16384
_K = 3
_H = 2048
_N = _B * _K

_info = plsc.get_sparse_core_info()
_NC, _NS, _L = _info.num_cores, _info.num_subcores, _info.num_lanes
_NW = _NC * _NS
_BPW = _N // _NW
_G = 16
_NB = _BPW // _G

_mesh = plsc.VectorSubcoreMesh(core_axis_name="c", subcore_axis_name="s")


@functools.partial(
    pl.kernel,
    mesh=_mesh,
    out_type=jax.ShapeDtypeStruct((_N, _H), jnp.float32),
    scratch_types=[
        pltpu.VMEM((_BPW,), jnp.int32),
        pltpu.VMEM((3, _H), jnp.float32),
        pltpu.VMEM((_G, _H), jnp.float32),
        pltpu.SemaphoreType.DMA,
        pltpu.SemaphoreType.DMA,
    ],
)
def _lookup(idx_hbm, table_hbm, out_hbm, idx_v, table_v, dummy_v, sem0, sem1):
    wid = lax.axis_index("s") * _NC + lax.axis_index("c")
    base = wid * _BPW
    pltpu.sync_copy(table_hbm, table_v)
    pltpu.sync_copy(idx_hbm.at[pl.ds(base, _BPW)], idx_v)

    def _issue(b, sem):
        v = idx_v[pl.ds(b * _G, _G)]
        v = jnp.where(v < 0, 2, v)
        for j in range(_G):
            r = v[j]
            pltpu.async_copy(
                table_v.at[pl.ds(r, 1)],
                out_hbm.at[pl.ds(base + b * _G + j, 1)],
                sem,
            )

    def _drain(sem):
        pltpu.make_async_copy(out_hbm.at[pl.ds(0, _G)], dummy_v, sem).wait()

    _issue(0, sem0)
    _issue(1, sem1)

    def _body(q, carry):
        b = 2 * q
        _drain(sem0)
        _issue(b + 2, sem0)
        _drain(sem1)
        _issue(b + 3, sem1)
        return carry

    lax.fori_loop(0, _NB // 2 - 1, _body, 0)
    _drain(sem0)
    _drain(sem1)


def kernel(arm_labels, embedding_weight):
    idx = arm_labels.reshape(_N).astype(jnp.int32)
    out = _lookup(idx, embedding_weight)
    return out.reshape(_B, _K, _H)
